# baseline (speedup 1.0000x reference)
import jax
import jax.numpy as jnp
from jax import lax
from jax.experimental import pallas as pl
from jax.experimental.pallas import tpu as pltpu
from jax.experimental.pallas import tpu_sc as plsc

B, C, H, W = 2, 3, 256, 256
F = 5
F2 = F * F
L = 16
RB = 8
NBLK = H // RB
NTASK = C * NBLK
TPW = NTASK // 16
NV = RB * W // L

_MESH = dict(core_axis_name="c", subcore_axis_name="s", num_cores=2,
             num_subcores=16)


def _sc_body(inp, vert, horiz, offx, offy, mask, out,
             plane, dxb, dyb, mb, vb, hb, accb, sems):
    b = lax.axis_index("c")
    s = lax.axis_index("s")
    xiota = lax.iota(jnp.int32, L).astype(jnp.float32)

    def issue(t, blk, slot):
        sem = sems.at[slot]
        r = blk * RB
        pltpu.async_copy(offy.at[b, t, pl.ds(r, RB)], dxb.at[slot], sem)
        pltpu.async_copy(offx.at[b, t, pl.ds(r, RB)], dyb.at[slot], sem)
        pltpu.async_copy(mask.at[b, t, pl.ds(r, RB)], mb.at[slot], sem)

    def drain(slot):
        sem = sems.at[slot]
        pltpu.make_async_copy(offy.at[0, 0, pl.ds(0, RB)], dxb.at[slot],
                              sem).wait()
        pltpu.make_async_copy(offx.at[0, 0, pl.ds(0, RB)], dyb.at[slot],
                              sem).wait()
        pltpu.make_async_copy(mask.at[0, 0, pl.ds(0, RB)], mb.at[slot],
                              sem).wait()

    def run_task(task, load_plane):
        ch = task // NBLK
        blk = task % NBLK
        r = blk * RB

        @pl.when(load_plane)
        def _():
            pltpu.sync_copy(inp.at[b, ch], plane)

        pltpu.sync_copy(vert.at[b, :, pl.ds(r, RB)], vb)
        pltpu.sync_copy(horiz.at[b, :, pl.ds(r, RB)], hb)
        blk8f = (r).astype(jnp.float32)

        issue(0, blk, 0)
        for t in range(F2):
            ti, tj = t // F, t % F
            slot = t & 1
            if t + 1 < F2:
                issue(t + 1, blk, 1 - slot)
            drain(slot)
            cyc = blk8f + (float(ti) - 1.5)
            cxc = float(tj) - 1.5

            def vec_body(v, carry, _t=t, _ti=ti, _tj=tj, _slot=slot,
                         _cyc=cyc, _cxc=cxc):
                row = v // L
                xb = (v % L) * L
                sl = pl.ds(xb, L)
                dx = dxb[_slot, row, sl]
                dy = dyb[_slot, row, sl]
                m = mb[_slot, row, sl]
                vt = vb[_ti, row, sl]
                ht = hb[_tj, row, sl]
                xoff = xiota + (xb.astype(jnp.float32) + _cxc)
                cy = _cyc + row.astype(jnp.float32)
                ix = jnp.clip(dx + xoff, 0.0, W - 1.0)
                iy = jnp.clip(dy + cy, 0.0, H - 1.0)
                x0 = ix.astype(jnp.int32)
                y0 = iy.astype(jnp.int32)
                wx1 = ix - x0.astype(jnp.float32)
                wy1 = iy - y0.astype(jnp.float32)
                x1 = jnp.minimum(x0 + 1, W - 1)
                y1 = jnp.minimum(y0 + 1, H - 1)
                v00 = plsc.load_gather(plane, [y0, x0])
                v01 = plsc.load_gather(plane, [y0, x1])
                v10 = plsc.load_gather(plane, [y1, x0])
                v11 = plsc.load_gather(plane, [y1, x1])
                l0 = v00 + wx1 * (v01 - v00)
                l1 = v10 + wx1 * (v11 - v10)
                smp = l0 + wy1 * (l1 - l0)
                contrib = (vt * ht * m) * smp
                if _t == 0:
                    accb[row, sl] = contrib
                else:
                    accb[row, sl] = accb[row, sl] + contrib
                return carry

            lax.fori_loop(0, NV, vec_body, jnp.int32(0))

        pltpu.sync_copy(accb, out.at[b, ch, pl.ds(r, RB)])

    def task_body(k, carry):
        task = s * TPW + k
        load_plane = jnp.logical_or(
            k == 0, task // NBLK != (task - 1) // NBLK)
        run_task(task, load_plane)
        return carry

    lax.fori_loop(0, TPW, task_body, jnp.int32(0))


def _build_sc_call():
    return pl.kernel(
        _sc_body,
        out_type=jax.ShapeDtypeStruct((B, C, H, W), jnp.float32),
        mesh=plsc.VectorSubcoreMesh(**_MESH),
        scratch_types=[
            pltpu.VMEM((H, W), jnp.float32),
            pltpu.VMEM((2, RB, W), jnp.float32),
            pltpu.VMEM((2, RB, W), jnp.float32),
            pltpu.VMEM((2, RB, W), jnp.float32),
            pltpu.VMEM((F, RB, W), jnp.float32),
            pltpu.VMEM((F, RB, W), jnp.float32),
            pltpu.VMEM((RB, W), jnp.float32),
            pltpu.SemaphoreType.DMA((2,)),
        ],
        compiler_params=pltpu.CompilerParams(use_tc_tiling_on_sc=True,
                                             needs_layout_passes=False),
    )


def kernel(tensorInput, tensorVertical, tensorHorizontal,
           tensorOffsetX, tensorOffsetY, tensorMask):
    return _build_sc_call()(tensorInput, tensorVertical, tensorHorizontal,
                            tensorOffsetX, tensorOffsetY, tensorMask)


# speedup vs baseline: 1.5072x; 1.5072x over previous
import jax
import jax.numpy as jnp
from jax import lax
from jax.experimental import pallas as pl
from jax.experimental.pallas import tpu as pltpu
from jax.experimental.pallas import tpu_sc as plsc

B, C, H, W = 2, 3, 256, 256
F = 5
F2 = F * F
L = 16
RB = 8
NBLK = H // RB
NTASK = C * NBLK
TPW = NTASK // 16
NV = RB * W // L
G = 3
GROUPS = [(g0, min(G, F2 - g0)) for g0 in range(0, F2, G)]

_MESH = dict(core_axis_name="c", subcore_axis_name="s", num_cores=2,
             num_subcores=16)


def _sc_body(inp, vert, horiz, offx, offy, mask, out,
             plane, dxb, dyb, mb, vb, hb, accb, sems):
    b = lax.axis_index("c")
    s = lax.axis_index("s")
    xiota = lax.iota(jnp.int32, L).astype(jnp.float32)

    def issue(g0, gn, blk, slot):
        sem = sems.at[slot]
        r = blk * RB
        gs = pl.ds(g0, gn)
        ds_ = pl.ds(0, gn)
        pltpu.async_copy(offy.at[b, gs, pl.ds(r, RB)], dxb.at[slot, ds_], sem)
        pltpu.async_copy(offx.at[b, gs, pl.ds(r, RB)], dyb.at[slot, ds_], sem)
        pltpu.async_copy(mask.at[b, gs, pl.ds(r, RB)], mb.at[slot, ds_], sem)

    def drain(gn, slot):
        sem = sems.at[slot]
        gs = pl.ds(0, gn)
        ds_ = pl.ds(0, gn)
        pltpu.make_async_copy(offy.at[0, gs, pl.ds(0, RB)],
                              dxb.at[slot, ds_], sem).wait()
        pltpu.make_async_copy(offx.at[0, gs, pl.ds(0, RB)],
                              dyb.at[slot, ds_], sem).wait()
        pltpu.make_async_copy(mask.at[0, gs, pl.ds(0, RB)],
                              mb.at[slot, ds_], sem).wait()

    def run_task(task, load_plane):
        ch = task // NBLK
        blk = task % NBLK
        r = blk * RB

        @pl.when(load_plane)
        def _():
            pltpu.sync_copy(inp.at[b, ch], plane)

        issue(*GROUPS[0], blk, 0)
        pltpu.sync_copy(vert.at[b, :, pl.ds(r, RB)], vb)
        pltpu.sync_copy(horiz.at[b, :, pl.ds(r, RB)], hb)
        rf = r.astype(jnp.float32)

        for gi, (g0, gn) in enumerate(GROUPS):
            slot = gi & 1
            if gi + 1 < len(GROUPS):
                issue(*GROUPS[gi + 1], blk, 1 - slot)
            drain(gn, slot)

            def vec_body(v, carry, _gi=gi, _g0=g0, _gn=gn, _slot=slot):
                row = v // L
                xb = (v % L) * L
                sl = pl.ds(xb, L)
                rowf = rf + row.astype(jnp.float32)
                xof = xiota + (xb.astype(jnp.float32) - 1.5)
                acc = None if _gi == 0 else accb[row, sl]
                for j in range(_gn):
                    t = _g0 + j
                    ti, tj = t // F, t % F
                    dx = dxb[_slot, j, row, sl]
                    dy = dyb[_slot, j, row, sl]
                    m = mb[_slot, j, row, sl]
                    vt = vb[ti, row, sl]
                    ht = hb[tj, row, sl]
                    ix = jnp.clip(dx + (xof + float(tj)), 0.0, W - 1.0)
                    iy = jnp.clip(dy + (rowf + (float(ti) - 1.5)),
                                  0.0, H - 1.0)
                    x0 = ix.astype(jnp.int32)
                    y0 = iy.astype(jnp.int32)
                    wx1 = ix - x0.astype(jnp.float32)
                    wy1 = iy - y0.astype(jnp.float32)
                    x1 = jnp.minimum(x0 + 1, W - 1)
                    y1 = jnp.minimum(y0 + 1, H - 1)
                    v00 = plsc.load_gather(plane, [y0, x0])
                    v01 = plsc.load_gather(plane, [y0, x1])
                    v10 = plsc.load_gather(plane, [y1, x0])
                    v11 = plsc.load_gather(plane, [y1, x1])
                    l0 = v00 + wx1 * (v01 - v00)
                    l1 = v10 + wx1 * (v11 - v10)
                    smp = l0 + wy1 * (l1 - l0)
                    contrib = (vt * ht * m) * smp
                    acc = contrib if acc is None else acc + contrib
                accb[row, sl] = acc
                return carry

            lax.fori_loop(0, NV, vec_body, jnp.int32(0))

        pltpu.sync_copy(accb, out.at[b, ch, pl.ds(r, RB)])

    def task_body(k, carry):
        task = s * TPW + k
        load_plane = jnp.logical_or(
            k == 0, task // NBLK != (task - 1) // NBLK)
        run_task(task, load_plane)
        return carry

    lax.fori_loop(0, TPW, task_body, jnp.int32(0))


def _build_sc_call():
    return pl.kernel(
        _sc_body,
        out_type=jax.ShapeDtypeStruct((B, C, H, W), jnp.float32),
        mesh=plsc.VectorSubcoreMesh(**_MESH),
        scratch_types=[
            pltpu.VMEM((H, W), jnp.float32),
            pltpu.VMEM((2, G, RB, W), jnp.float32),
            pltpu.VMEM((2, G, RB, W), jnp.float32),
            pltpu.VMEM((2, G, RB, W), jnp.float32),
            pltpu.VMEM((F, RB, W), jnp.float32),
            pltpu.VMEM((F, RB, W), jnp.float32),
            pltpu.VMEM((RB, W), jnp.float32),
            pltpu.SemaphoreType.DMA((2,)),
        ],
        compiler_params=pltpu.CompilerParams(use_tc_tiling_on_sc=True,
                                             needs_layout_passes=False),
    )


def kernel(tensorInput, tensorVertical, tensorHorizontal,
           tensorOffsetX, tensorOffsetY, tensorMask):
    return _build_sc_call()(tensorInput, tensorVertical, tensorHorizontal,
                            tensorOffsetX, tensorOffsetY, tensorMask)


# speedup vs baseline: 1.5371x; 1.0198x over previous
import jax
import jax.numpy as jnp
from jax import lax
from jax.experimental import pallas as pl
from jax.experimental.pallas import tpu as pltpu
from jax.experimental.pallas import tpu_sc as plsc

B, C, H, W = 2, 3, 256, 256
F = 5
F2 = F * F
L = 16
RB = 8
NBLK = H // RB
NTASK = C * NBLK
TPW = NTASK // 16
NV = RB * W // L
G = 3
GROUPS = [(g0, min(G, F2 - g0)) for g0 in range(0, F2, G)]

_MESH = dict(core_axis_name="c", subcore_axis_name="s", num_cores=2,
             num_subcores=16)


def _sc_body(inp, vert, horiz, offx, offy, mask, out,
             plane, dxb, dyb, mb, vb, hb, accb, sems):
    b = lax.axis_index("c")
    s = lax.axis_index("s")
    xiota = lax.iota(jnp.int32, L).astype(jnp.float32)

    def issue(g0, gn, blk, slot):
        sem = sems.at[slot]
        r = blk * RB
        gs = pl.ds(g0, gn)
        ds_ = pl.ds(0, gn)
        pltpu.async_copy(offy.at[b, gs, pl.ds(r, RB)], dxb.at[slot, ds_], sem)
        pltpu.async_copy(offx.at[b, gs, pl.ds(r, RB)], dyb.at[slot, ds_], sem)
        pltpu.async_copy(mask.at[b, gs, pl.ds(r, RB)], mb.at[slot, ds_], sem)

    def drain(gn, slot):
        sem = sems.at[slot]
        gs = pl.ds(0, gn)
        ds_ = pl.ds(0, gn)
        pltpu.make_async_copy(offy.at[0, gs, pl.ds(0, RB)],
                              dxb.at[slot, ds_], sem).wait()
        pltpu.make_async_copy(offx.at[0, gs, pl.ds(0, RB)],
                              dyb.at[slot, ds_], sem).wait()
        pltpu.make_async_copy(mask.at[0, gs, pl.ds(0, RB)],
                              mb.at[slot, ds_], sem).wait()

    def run_task(task, load_plane):
        ch = task // NBLK
        blk = task % NBLK
        r = blk * RB

        @pl.when(load_plane)
        def _():
            pltpu.sync_copy(inp.at[b, ch], plane)

        issue(*GROUPS[0], blk, 0)
        pltpu.sync_copy(vert.at[b, :, pl.ds(r, RB)], vb)
        pltpu.sync_copy(horiz.at[b, :, pl.ds(r, RB)], hb)
        rf = r.astype(jnp.float32)

        for gi, (g0, gn) in enumerate(GROUPS):
            slot = gi & 1
            if gi + 1 < len(GROUPS):
                issue(*GROUPS[gi + 1], blk, 1 - slot)
            drain(gn, slot)

            def vec_body(v, _gi=gi, _g0=g0, _gn=gn, _slot=slot):
                row = v // L
                xb = (v % L) * L
                sl = pl.ds(xb, L)
                rowf = rf + row.astype(jnp.float32)
                xof = xiota + (xb.astype(jnp.float32) - 1.5)
                acc = None if _gi == 0 else accb[row, sl]
                for j in range(_gn):
                    t = _g0 + j
                    ti, tj = t // F, t % F
                    dx = dxb[_slot, j, row, sl]
                    dy = dyb[_slot, j, row, sl]
                    m = mb[_slot, j, row, sl]
                    vt = vb[ti, row, sl]
                    ht = hb[tj, row, sl]
                    ix = jnp.clip(dx + (xof + float(tj)), 0.0, W - 1.0)
                    iy = jnp.clip(dy + (rowf + (float(ti) - 1.5)),
                                  0.0, H - 1.0)
                    x0 = ix.astype(jnp.int32)
                    y0 = iy.astype(jnp.int32)
                    wx1 = ix - x0.astype(jnp.float32)
                    wy1 = iy - y0.astype(jnp.float32)
                    x1 = jnp.minimum(x0 + 1, W - 1)
                    y1 = jnp.minimum(y0 + 1, H - 1)
                    v00 = plsc.load_gather(plane, [y0, x0])
                    v01 = plsc.load_gather(plane, [y0, x1])
                    v10 = plsc.load_gather(plane, [y1, x0])
                    v11 = plsc.load_gather(plane, [y1, x1])
                    l0 = v00 + wx1 * (v01 - v00)
                    l1 = v10 + wx1 * (v11 - v10)
                    smp = l0 + wy1 * (l1 - l0)
                    contrib = (vt * ht * m) * smp
                    acc = contrib if acc is None else acc + contrib
                accb[row, sl] = acc

            plsc.parallel_loop(0, NV, 1, unroll=2)(vec_body)

        pltpu.sync_copy(accb, out.at[b, ch, pl.ds(r, RB)])

    def task_body(k, carry):
        task = s * TPW + k
        load_plane = jnp.logical_or(
            k == 0, task // NBLK != (task - 1) // NBLK)
        run_task(task, load_plane)
        return carry

    lax.fori_loop(0, TPW, task_body, jnp.int32(0))


def _build_sc_call():
    return pl.kernel(
        _sc_body,
        out_type=jax.ShapeDtypeStruct((B, C, H, W), jnp.float32),
        mesh=plsc.VectorSubcoreMesh(**_MESH),
        scratch_types=[
            pltpu.VMEM((H, W), jnp.float32),
            pltpu.VMEM((2, G, RB, W), jnp.float32),
            pltpu.VMEM((2, G, RB, W), jnp.float32),
            pltpu.VMEM((2, G, RB, W), jnp.float32),
            pltpu.VMEM((F, RB, W), jnp.float32),
            pltpu.VMEM((F, RB, W), jnp.float32),
            pltpu.VMEM((RB, W), jnp.float32),
            pltpu.SemaphoreType.DMA((2,)),
        ],
        compiler_params=pltpu.CompilerParams(use_tc_tiling_on_sc=True,
                                             needs_layout_passes=False),
    )


def kernel(tensorInput, tensorVertical, tensorHorizontal,
           tensorOffsetX, tensorOffsetY, tensorMask):
    return _build_sc_call()(tensorInput, tensorVertical, tensorHorizontal,
                            tensorOffsetX, tensorOffsetY, tensorMask)


# speedup vs baseline: 2.0491x; 1.3332x over previous
import jax
import jax.numpy as jnp
from jax import lax
from jax.experimental import pallas as pl
from jax.experimental.pallas import tpu as pltpu
from jax.experimental.pallas import tpu_sc as plsc

B, C, H, W = 2, 3, 256, 256
F = 5
F2 = F * F
L = 16
RB = 4
NBLK = H // RB
TPW = NBLK // 16
NV = RB * W // L
G = 2
GROUPS = [(g0, min(G, F2 - g0)) for g0 in range(0, F2, G)]
MASKHI = -65536

_MESH = dict(core_axis_name="c", subcore_axis_name="s", num_cores=2,
             num_subcores=16)


def _hi_f32(w):
    return plsc.bitcast(w & MASKHI, jnp.float32)


def _lo_f32(w):
    return plsc.bitcast(w << 16, jnp.float32)


def _lerp2(w00, w01, w10, w11, wx1, wy1, extract):
    v00, v01, v10, v11 = extract(w00), extract(w01), extract(w10), extract(w11)
    l0 = v00 + wx1 * (v01 - v00)
    l1 = v10 + wx1 * (v11 - v10)
    return l0 + wy1 * (l1 - l0)


def _sc_body(p01, p2, vert, horiz, offx, offy, mask, out,
             plane01, plane2, dxb, dyb, mb, vb, hb, accb, sems):
    b = lax.axis_index("c")
    s = lax.axis_index("s")
    xiota = lax.iota(jnp.int32, L).astype(jnp.float32)

    def issue(g0, gn, blk, slot):
        sem = sems.at[slot]
        r = blk * RB
        gs = pl.ds(g0, gn)
        ds_ = pl.ds(0, gn)
        pltpu.async_copy(offy.at[b, gs, pl.ds(r, RB)], dxb.at[slot, ds_], sem)
        pltpu.async_copy(offx.at[b, gs, pl.ds(r, RB)], dyb.at[slot, ds_], sem)
        pltpu.async_copy(mask.at[b, gs, pl.ds(r, RB)], mb.at[slot, ds_], sem)

    def drain(gn, slot):
        sem = sems.at[slot]
        gs = pl.ds(0, gn)
        ds_ = pl.ds(0, gn)
        pltpu.make_async_copy(offy.at[0, gs, pl.ds(0, RB)],
                              dxb.at[slot, ds_], sem).wait()
        pltpu.make_async_copy(offx.at[0, gs, pl.ds(0, RB)],
                              dyb.at[slot, ds_], sem).wait()
        pltpu.make_async_copy(mask.at[0, gs, pl.ds(0, RB)],
                              mb.at[slot, ds_], sem).wait()

    def run_task(blk):
        r = blk * RB
        issue(*GROUPS[0], blk, 0)
        pltpu.sync_copy(vert.at[b, :, pl.ds(r, RB)], vb)
        pltpu.sync_copy(horiz.at[b, :, pl.ds(r, RB)], hb)
        rf = r.astype(jnp.float32)

        for gi, (g0, gn) in enumerate(GROUPS):
            slot = gi & 1
            if gi + 1 < len(GROUPS):
                issue(*GROUPS[gi + 1], blk, 1 - slot)
            drain(gn, slot)

            def vec_body(v, _gi=gi, _g0=g0, _gn=gn, _slot=slot):
                row = v // L
                xb = (v % L) * L
                sl = pl.ds(xb, L)
                rowf = rf + row.astype(jnp.float32)
                xof = xiota + (xb.astype(jnp.float32) - 1.5)
                if _gi == 0:
                    acc0 = acc1 = acc2 = None
                else:
                    acc0 = accb[0, row, sl]
                    acc1 = accb[1, row, sl]
                    acc2 = accb[2, row, sl]
                for j in range(_gn):
                    t = _g0 + j
                    ti, tj = t // F, t % F
                    dx = dxb[_slot, j, row, sl]
                    dy = dyb[_slot, j, row, sl]
                    m = mb[_slot, j, row, sl]
                    vt = vb[ti, row, sl]
                    ht = hb[tj, row, sl]
                    ix = jnp.clip(dx + (xof + float(tj)), 0.0, W - 1.0)
                    iy = jnp.clip(dy + (rowf + (float(ti) - 1.5)),
                                  0.0, H - 1.0)
                    x0 = ix.astype(jnp.int32)
                    y0 = iy.astype(jnp.int32)
                    wx1 = ix - x0.astype(jnp.float32)
                    wy1 = iy - y0.astype(jnp.float32)
                    x1 = jnp.minimum(x0 + 1, W - 1)
                    y1 = jnp.minimum(y0 + 1, H - 1)
                    w00 = plsc.load_gather(plane01, [y0, x0])
                    w01 = plsc.load_gather(plane01, [y0, x1])
                    w10 = plsc.load_gather(plane01, [y1, x0])
                    w11 = plsc.load_gather(plane01, [y1, x1])
                    x0h = x0 >> 1
                    x1h = x1 >> 1
                    u00 = plsc.load_gather(plane2, [y0, x0h])
                    u01 = plsc.load_gather(plane2, [y0, x1h])
                    u10 = plsc.load_gather(plane2, [y1, x0h])
                    u11 = plsc.load_gather(plane2, [y1, x1h])
                    s0 = _lerp2(w00, w01, w10, w11, wx1, wy1, _lo_f32)
                    s1 = _lerp2(w00, w01, w10, w11, wx1, wy1, _hi_f32)
                    p0 = (x0 & 1) == 1
                    p1 = (x1 & 1) == 1

                    def _sel(p):
                        return lambda w: jnp.where(p, _hi_f32(w), _lo_f32(w))

                    e0, e1 = _sel(p0), _sel(p1)
                    c2_00, c2_01 = e0(u00), e1(u01)
                    c2_10, c2_11 = e0(u10), e1(u11)
                    l0 = c2_00 + wx1 * (c2_01 - c2_00)
                    l1 = c2_10 + wx1 * (c2_11 - c2_10)
                    s2 = l0 + wy1 * (l1 - l0)
                    wgt = (vt * ht) * m
                    d0, d1, d2 = wgt * s0, wgt * s1, wgt * s2
                    if acc0 is None:
                        acc0, acc1, acc2 = d0, d1, d2
                    else:
                        acc0, acc1, acc2 = acc0 + d0, acc1 + d1, acc2 + d2
                accb[0, row, sl] = acc0
                accb[1, row, sl] = acc1
                accb[2, row, sl] = acc2

            plsc.parallel_loop(0, NV, 1, unroll=2)(vec_body)

        pltpu.sync_copy(accb, out.at[b, :, pl.ds(r, RB)])

    pltpu.sync_copy(p01.at[b], plane01)
    pltpu.sync_copy(p2.at[b], plane2)

    def task_body(k, carry):
        run_task(s * TPW + k)
        return carry

    lax.fori_loop(0, TPW, task_body, jnp.int32(0))


def _build_sc_call():
    return pl.kernel(
        _sc_body,
        out_type=jax.ShapeDtypeStruct((B, C, H, W), jnp.float32),
        mesh=plsc.VectorSubcoreMesh(**_MESH),
        scratch_types=[
            pltpu.VMEM((H, W), jnp.int32),
            pltpu.VMEM((H, W // 2), jnp.int32),
            pltpu.VMEM((2, G, RB, W), jnp.float32),
            pltpu.VMEM((2, G, RB, W), jnp.float32),
            pltpu.VMEM((2, G, RB, W), jnp.float32),
            pltpu.VMEM((F, RB, W), jnp.float32),
            pltpu.VMEM((F, RB, W), jnp.float32),
            pltpu.VMEM((C, RB, W), jnp.float32),
            pltpu.SemaphoreType.DMA((2,)),
        ],
        compiler_params=pltpu.CompilerParams(use_tc_tiling_on_sc=True,
                                             needs_layout_passes=False),
    )


def kernel(tensorInput, tensorVertical, tensorHorizontal,
           tensorOffsetX, tensorOffsetY, tensorMask):
    u = lax.bitcast_convert_type(
        tensorInput.astype(jnp.bfloat16), jnp.uint16).astype(jnp.uint32)
    c01 = lax.bitcast_convert_type(u[:, 0] | (u[:, 1] << 16), jnp.int32)
    u2 = u[:, 2].reshape(B, H, W // 2, 2)
    c2 = lax.bitcast_convert_type(u2[..., 0] | (u2[..., 1] << 16), jnp.int32)
    return _build_sc_call()(c01, c2, tensorVertical, tensorHorizontal,
                            tensorOffsetX, tensorOffsetY, tensorMask)
